# baseline (device time: 14205 ns/iter reference)
import os

import jax
import jax.numpy as jnp
from jax import lax
from jax.experimental import pallas as pl
from jax.experimental.pallas import tpu as pltpu

N_DEV = 8

_VARIANT = os.environ.get("KERNEL_VARIANT", "full")


def kernel(x):
    m_per, n = x.shape

    if _VARIANT == "copyonly":
        def body_copy(x_ref, out_ref):
            out_ref[:, :] = x_ref[:2, :]
        return pl.pallas_call(
            body_copy,
            out_shape=jax.ShapeDtypeStruct((2, n), jnp.float32),
            in_specs=[pl.BlockSpec(memory_space=pltpu.VMEM)],
            out_specs=pl.BlockSpec(memory_space=pltpu.VMEM),
        )(x)

    if _VARIANT == "nocomm":
        def body_nc(x_ref, out_ref):
            my = lax.axis_index("i")
            n_blk = m_per // 8

            def step(b, carry):
                m, bidx = carry
                blk = x_ref[pl.ds(b * 8, 8), :]
                take = blk > m
                return (jnp.where(take, blk, m), jnp.where(take, b, bidx))

            m0 = jnp.full((8, n), -jnp.inf, jnp.float32)
            b0 = jnp.zeros((8, n), jnp.int32)
            m, bidx = lax.fori_loop(0, n_blk, step, (m0, b0), unroll=8)
            local_max = jnp.max(m, axis=0)
            sub = lax.broadcasted_iota(jnp.int32, (8, n), 0)
            rows = bidx * 8 + sub
            cand = jnp.where(m == local_max[None, :], rows, jnp.int32(m_per))
            out_ref[0, :] = local_max
            out_ref[1, :] = jnp.min(cand, axis=0).astype(jnp.float32) + (
                my.astype(jnp.float32) * jnp.float32(m_per)
            )
        return pl.pallas_call(
            body_nc,
            out_shape=jax.ShapeDtypeStruct((2, n), jnp.float32),
            in_specs=[pl.BlockSpec(memory_space=pltpu.VMEM)],
            out_specs=pl.BlockSpec(memory_space=pltpu.VMEM),
        )(x)

    C = 4
    cw = n // C

    def body(x_ref, out_ref, xbuf, comm_ref, copy_sems, send_sems, recv_sems):
        my = lax.axis_index("i")

        copies = []
        for c in range(C):
            cp = pltpu.make_async_copy(
                x_ref.at[:, pl.ds(c * cw, cw)], xbuf.at[c], copy_sems.at[c]
            )
            cp.start()
            copies.append(cp)

        barrier_sem = pltpu.get_barrier_semaphore()
        for d in range(1, N_DEV):
            peer = lax.rem(my + d, N_DEV)
            pl.semaphore_signal(
                barrier_sem, inc=1,
                device_id=(peer,), device_id_type=pl.DeviceIdType.MESH,
            )

        def reduce_chunk(c):
            def step(b, carry):
                m, bidx = carry
                blk = xbuf[c, pl.ds(b * 8, 8), :]
                take = blk > m
                return (jnp.where(take, blk, m), jnp.where(take, b, bidx))

            m0 = jnp.full((8, cw), -jnp.inf, jnp.float32)
            b0 = jnp.zeros((8, cw), jnp.int32)
            m, bidx = lax.fori_loop(0, m_per // 8, step, (m0, b0), unroll=8)

            local_max = jnp.max(m, axis=0)
            sub = lax.broadcasted_iota(jnp.int32, (8, cw), 0)
            rows = bidx * 8 + sub
            cand = jnp.where(m == local_max[None, :], rows, jnp.int32(m_per))
            local_idx = jnp.min(cand, axis=0).astype(jnp.float32) + (
                my.astype(jnp.float32) * jnp.float32(m_per)
            )
            comm_ref[c, my, 0, :] = local_max
            comm_ref[c, my, 1, :] = local_idx

        def send_chunk(c):
            for d in range(1, N_DEV):
                peer = lax.rem(my + d, N_DEV)
                pltpu.make_async_remote_copy(
                    src_ref=comm_ref.at[c, my],
                    dst_ref=comm_ref.at[c, my],
                    send_sem=send_sems.at[c, peer],
                    recv_sem=recv_sems.at[c, my],
                    device_id=(peer,),
                    device_id_type=pl.DeviceIdType.MESH,
                ).start()

        copies[0].wait()
        reduce_chunk(0)
        pl.semaphore_wait(barrier_sem, N_DEV - 1)
        send_chunk(0)
        for c in range(1, C):
            copies[c].wait()
            reduce_chunk(c)
            send_chunk(c)

        for c in range(C):
            for d in range(1, N_DEV):
                peer = lax.rem(my + d, N_DEV)
                pltpu.make_async_remote_copy(
                    src_ref=comm_ref.at[c, peer],
                    dst_ref=comm_ref.at[c, peer],
                    send_sem=send_sems.at[c, peer],
                    recv_sem=recv_sems.at[c, peer],
                    device_id=(peer,),
                    device_id_type=pl.DeviceIdType.MESH,
                ).wait_recv()

        for c in range(C):
            vals = comm_ref[c, :, 0, :]
            idxs = comm_ref[c, :, 1, :]
            gmax = jnp.max(vals, axis=0)
            gidx = jnp.min(
                jnp.where(vals == gmax[None, :], idxs, jnp.float32(jnp.inf)),
                axis=0,
            )
            out_ref[0, pl.ds(c * cw, cw)] = gmax
            out_ref[1, pl.ds(c * cw, cw)] = gidx

        for c in range(C):
            for d in range(1, N_DEV):
                peer = lax.rem(my + d, N_DEV)
                pltpu.make_async_remote_copy(
                    src_ref=comm_ref.at[c, my],
                    dst_ref=comm_ref.at[c, my],
                    send_sem=send_sems.at[c, peer],
                    recv_sem=recv_sems.at[c, peer],
                    device_id=(peer,),
                    device_id_type=pl.DeviceIdType.MESH,
                ).wait_send()

    return pl.pallas_call(
        body,
        out_shape=jax.ShapeDtypeStruct((2, n), jnp.float32),
        in_specs=[pl.BlockSpec(memory_space=pl.ANY)],
        out_specs=pl.BlockSpec(memory_space=pltpu.VMEM),
        scratch_shapes=[
            pltpu.VMEM((C, m_per, cw), jnp.float32),
            pltpu.VMEM((C, N_DEV, 2, cw), jnp.float32),
            pltpu.SemaphoreType.DMA((C,)),
            pltpu.SemaphoreType.DMA((C, N_DEV)),
            pltpu.SemaphoreType.DMA((C, N_DEV)),
        ],
        compiler_params=pltpu.CompilerParams(collective_id=0),
    )(x)


# device time: 11916 ns/iter; 1.1921x vs baseline; 1.1921x over previous
import os

import jax
import jax.numpy as jnp
from jax import lax
from jax.experimental import pallas as pl
from jax.experimental.pallas import tpu as pltpu

N_DEV = 8

_VARIANT = os.environ.get("KERNEL_VARIANT", "full")


def kernel(x):
    m_per, n = x.shape

    if _VARIANT == "copyonly":
        def body_copy(x_ref, out_ref):
            out_ref[:, :] = x_ref[:2, :]
        return pl.pallas_call(
            body_copy,
            out_shape=jax.ShapeDtypeStruct((2, n), jnp.float32),
            in_specs=[pl.BlockSpec(memory_space=pltpu.VMEM)],
            out_specs=pl.BlockSpec(memory_space=pltpu.VMEM),
        )(x)

    if _VARIANT == "nocomm":
        def body_nc(x_ref, out_ref):
            my = lax.axis_index("i")
            n_blk = m_per // 8

            def step(b, carry):
                m, bidx = carry
                blk = x_ref[pl.ds(b * 8, 8), :]
                take = blk > m
                return (jnp.where(take, blk, m), jnp.where(take, b, bidx))

            m0 = jnp.full((8, n), -jnp.inf, jnp.float32)
            b0 = jnp.zeros((8, n), jnp.int32)
            m, bidx = lax.fori_loop(0, n_blk, step, (m0, b0), unroll=8)
            local_max = jnp.max(m, axis=0)
            sub = lax.broadcasted_iota(jnp.int32, (8, n), 0)
            rows = bidx * 8 + sub
            cand = jnp.where(m == local_max[None, :], rows, jnp.int32(m_per))
            out_ref[0, :] = local_max
            out_ref[1, :] = jnp.min(cand, axis=0).astype(jnp.float32) + (
                my.astype(jnp.float32) * jnp.float32(m_per)
            )
        return pl.pallas_call(
            body_nc,
            out_shape=jax.ShapeDtypeStruct((2, n), jnp.float32),
            in_specs=[pl.BlockSpec(memory_space=pltpu.VMEM)],
            out_specs=pl.BlockSpec(memory_space=pltpu.VMEM),
        )(x)

    C = 4
    rows_per = m_per // C

    def body(x_ref, out_ref, xbuf, comm_ref, copy_sems, send_sems, recv_sems):
        my = lax.axis_index("i")

        copies = []
        for c in range(C):
            cp = pltpu.make_async_copy(
                x_ref.at[pl.ds(c * rows_per, rows_per), :],
                xbuf.at[c],
                copy_sems.at[c],
            )
            cp.start()
            copies.append(cp)

        barrier_sem = pltpu.get_barrier_semaphore()
        for d in range(1, N_DEV):
            peer = lax.rem(my + d, N_DEV)
            pl.semaphore_signal(
                barrier_sem, inc=1,
                device_id=(peer,), device_id_type=pl.DeviceIdType.MESH,
            )

        m = jnp.full((8, n), -jnp.inf, jnp.float32)
        bidx = jnp.zeros((8, n), jnp.int32)
        for c in range(C):
            copies[c].wait()

            def step(b, carry, c=c):
                mm, bb = carry
                blk = xbuf[c, pl.ds(b * 8, 8), :]
                take = blk > mm
                return (
                    jnp.where(take, blk, mm),
                    jnp.where(take, b + c * (rows_per // 8), bb),
                )

            m, bidx = lax.fori_loop(
                0, rows_per // 8, step, (m, bidx), unroll=8
            )

        local_max = jnp.max(m, axis=0)
        sub = lax.broadcasted_iota(jnp.int32, (8, n), 0)
        rows = bidx * 8 + sub
        cand = jnp.where(m == local_max[None, :], rows, jnp.int32(m_per))
        local_idx = jnp.min(cand, axis=0).astype(jnp.float32) + (
            my.astype(jnp.float32) * jnp.float32(m_per)
        )
        comm_ref[my, 0, :] = local_max
        comm_ref[my, 1, :] = local_idx

        pl.semaphore_wait(barrier_sem, N_DEV - 1)

        for d in range(1, N_DEV):
            peer = lax.rem(my + d, N_DEV)
            pltpu.make_async_remote_copy(
                src_ref=comm_ref.at[my],
                dst_ref=comm_ref.at[my],
                send_sem=send_sems.at[peer],
                recv_sem=recv_sems.at[my],
                device_id=(peer,),
                device_id_type=pl.DeviceIdType.MESH,
            ).start()

        for d in range(1, N_DEV):
            peer = lax.rem(my + d, N_DEV)
            pltpu.make_async_remote_copy(
                src_ref=comm_ref.at[peer],
                dst_ref=comm_ref.at[peer],
                send_sem=send_sems.at[peer],
                recv_sem=recv_sems.at[peer],
                device_id=(peer,),
                device_id_type=pl.DeviceIdType.MESH,
            ).wait_recv()

        vals = comm_ref[:, 0, :]
        idxs = comm_ref[:, 1, :]
        gmax = jnp.max(vals, axis=0)
        gidx = jnp.min(
            jnp.where(vals == gmax[None, :], idxs, jnp.float32(jnp.inf)), axis=0
        )
        out_ref[0, :] = gmax
        out_ref[1, :] = gidx

        for d in range(1, N_DEV):
            peer = lax.rem(my + d, N_DEV)
            pltpu.make_async_remote_copy(
                src_ref=comm_ref.at[my],
                dst_ref=comm_ref.at[my],
                send_sem=send_sems.at[peer],
                recv_sem=recv_sems.at[peer],
                device_id=(peer,),
                device_id_type=pl.DeviceIdType.MESH,
            ).wait_send()

    return pl.pallas_call(
        body,
        out_shape=jax.ShapeDtypeStruct((2, n), jnp.float32),
        in_specs=[pl.BlockSpec(memory_space=pl.ANY)],
        out_specs=pl.BlockSpec(memory_space=pltpu.VMEM),
        scratch_shapes=[
            pltpu.VMEM((C, rows_per, n), jnp.float32),
            pltpu.VMEM((N_DEV, 2, n), jnp.float32),
            pltpu.SemaphoreType.DMA((C,)),
            pltpu.SemaphoreType.DMA((N_DEV,)),
            pltpu.SemaphoreType.DMA((N_DEV,)),
        ],
        compiler_params=pltpu.CompilerParams(collective_id=0),
    )(x)


# device time: 10751 ns/iter; 1.3213x vs baseline; 1.1084x over previous
import os

import jax
import jax.numpy as jnp
from jax import lax
from jax.experimental import pallas as pl
from jax.experimental.pallas import tpu as pltpu

N_DEV = 8

_VARIANT = os.environ.get("KERNEL_VARIANT", "full")


def kernel(x):
    m_per, n = x.shape

    if _VARIANT == "copyonly":
        def body_copy(x_ref, out_ref):
            out_ref[:, :] = x_ref[:2, :]
        return pl.pallas_call(
            body_copy,
            out_shape=jax.ShapeDtypeStruct((2, n), jnp.float32),
            in_specs=[pl.BlockSpec(memory_space=pltpu.VMEM)],
            out_specs=pl.BlockSpec(memory_space=pltpu.VMEM),
        )(x)

    if _VARIANT == "commonly":
        def body_co(x_ref, out_ref, comm_ref, send_sems, recv_sems):
            my = lax.axis_index("i")
            barrier_sem = pltpu.get_barrier_semaphore()
            for d in range(1, N_DEV):
                peer = lax.rem(my + d, N_DEV)
                pl.semaphore_signal(
                    barrier_sem, inc=1,
                    device_id=(peer,), device_id_type=pl.DeviceIdType.MESH,
                )
            comm_ref[my, 0, :] = jnp.ones((n,), jnp.float32)
            comm_ref[my, 1, :] = jnp.ones((n,), jnp.float32)
            pl.semaphore_wait(barrier_sem, N_DEV - 1)
            for d in range(1, N_DEV):
                peer = lax.rem(my + d, N_DEV)
                pltpu.make_async_remote_copy(
                    src_ref=comm_ref.at[my], dst_ref=comm_ref.at[my],
                    send_sem=send_sems.at[peer], recv_sem=recv_sems.at[my],
                    device_id=(peer,), device_id_type=pl.DeviceIdType.MESH,
                ).start()
            for d in range(1, N_DEV):
                peer = lax.rem(my + d, N_DEV)
                pltpu.make_async_remote_copy(
                    src_ref=comm_ref.at[peer], dst_ref=comm_ref.at[peer],
                    send_sem=send_sems.at[peer], recv_sem=recv_sems.at[peer],
                    device_id=(peer,), device_id_type=pl.DeviceIdType.MESH,
                ).wait_recv()
            vals = comm_ref[:, 0, :]
            idxs = comm_ref[:, 1, :]
            gmax = jnp.max(vals, axis=0)
            gidx = jnp.min(
                jnp.where(vals == gmax[None, :], idxs, jnp.float32(jnp.inf)),
                axis=0,
            )
            out_ref[0, :] = gmax
            out_ref[1, :] = gidx
            for d in range(1, N_DEV):
                peer = lax.rem(my + d, N_DEV)
                pltpu.make_async_remote_copy(
                    src_ref=comm_ref.at[my], dst_ref=comm_ref.at[my],
                    send_sem=send_sems.at[peer], recv_sem=recv_sems.at[peer],
                    device_id=(peer,), device_id_type=pl.DeviceIdType.MESH,
                ).wait_send()
        return pl.pallas_call(
            body_co,
            out_shape=jax.ShapeDtypeStruct((2, n), jnp.float32),
            in_specs=[pl.BlockSpec(memory_space=pl.ANY)],
            out_specs=pl.BlockSpec(memory_space=pltpu.VMEM),
            scratch_shapes=[
                pltpu.VMEM((N_DEV, 2, n), jnp.float32),
                pltpu.SemaphoreType.DMA((N_DEV,)),
                pltpu.SemaphoreType.DMA((N_DEV,)),
            ],
            compiler_params=pltpu.CompilerParams(collective_id=0),
        )(x)

    if _VARIANT == "nocomm":
        def body_nc(x_ref, out_ref):
            my = lax.axis_index("i")
            n_blk = m_per // 8

            def step(b, carry):
                m, bidx = carry
                blk = x_ref[pl.ds(b * 8, 8), :]
                take = blk > m
                return (jnp.where(take, blk, m), jnp.where(take, b, bidx))

            m0 = jnp.full((8, n), -jnp.inf, jnp.float32)
            b0 = jnp.zeros((8, n), jnp.int32)
            m, bidx = lax.fori_loop(0, n_blk, step, (m0, b0), unroll=8)
            local_max = jnp.max(m, axis=0)
            sub = lax.broadcasted_iota(jnp.int32, (8, n), 0)
            rows = bidx * 8 + sub
            cand = jnp.where(m == local_max[None, :], rows, jnp.int32(m_per))
            out_ref[0, :] = local_max
            out_ref[1, :] = jnp.min(cand, axis=0).astype(jnp.float32) + (
                my.astype(jnp.float32) * jnp.float32(m_per)
            )
        return pl.pallas_call(
            body_nc,
            out_shape=jax.ShapeDtypeStruct((2, n), jnp.float32),
            in_specs=[pl.BlockSpec(memory_space=pltpu.VMEM)],
            out_specs=pl.BlockSpec(memory_space=pltpu.VMEM),
        )(x)

    C = 4
    rows_per = m_per // C

    def body(x_ref, out_ref, xbuf, comm_ref, copy_sems, send_sems, recv_sems):
        my = lax.axis_index("i")

        copies = []
        for c in range(C):
            cp = pltpu.make_async_copy(
                x_ref.at[pl.ds(c * rows_per, rows_per), :],
                xbuf.at[c],
                copy_sems.at[c],
            )
            cp.start()
            copies.append(cp)

        barrier_sem = pltpu.get_barrier_semaphore()
        for d in range(1, N_DEV):
            peer = lax.rem(my + d, N_DEV)
            pl.semaphore_signal(
                barrier_sem, inc=1,
                device_id=(peer,), device_id_type=pl.DeviceIdType.MESH,
            )

        m = jnp.full((8, n), -jnp.inf, jnp.float32)
        bidx = jnp.zeros((8, n), jnp.int32)
        for c in range(C):
            copies[c].wait()

            def step(b, carry, c=c):
                mm, bb = carry
                blk = xbuf[c, pl.ds(b * 8, 8), :]
                take = blk > mm
                return (
                    jnp.where(take, blk, mm),
                    jnp.where(take, b + c * (rows_per // 8), bb),
                )

            m, bidx = lax.fori_loop(
                0, rows_per // 8, step, (m, bidx), unroll=8
            )

        local_max = jnp.max(m, axis=0)
        sub = lax.broadcasted_iota(jnp.int32, (8, n), 0)
        rows = bidx * 8 + sub
        cand = jnp.where(m == local_max[None, :], rows, jnp.int32(m_per))
        local_idx = jnp.min(cand, axis=0).astype(jnp.float32) + (
            my.astype(jnp.float32) * jnp.float32(m_per)
        )
        comm_ref[my, 0, :] = local_max
        comm_ref[my, 1, :] = local_idx

        pl.semaphore_wait(barrier_sem, N_DEV - 1)

        for d in range(1, N_DEV):
            peer = lax.rem(my + d, N_DEV)
            pltpu.make_async_remote_copy(
                src_ref=comm_ref.at[my],
                dst_ref=comm_ref.at[my],
                send_sem=send_sems.at[peer],
                recv_sem=recv_sems.at[my],
                device_id=(peer,),
                device_id_type=pl.DeviceIdType.MESH,
            ).start()

        for d in range(1, N_DEV):
            peer = lax.rem(my + d, N_DEV)
            pltpu.make_async_remote_copy(
                src_ref=comm_ref.at[peer],
                dst_ref=comm_ref.at[peer],
                send_sem=send_sems.at[peer],
                recv_sem=recv_sems.at[peer],
                device_id=(peer,),
                device_id_type=pl.DeviceIdType.MESH,
            ).wait_recv()

        vals = comm_ref[:, 0, :]
        idxs = comm_ref[:, 1, :]
        gmax = jnp.max(vals, axis=0)
        gidx = jnp.min(
            jnp.where(vals == gmax[None, :], idxs, jnp.float32(jnp.inf)), axis=0
        )
        out_ref[0, :] = gmax
        out_ref[1, :] = gidx

        for d in range(1, N_DEV):
            peer = lax.rem(my + d, N_DEV)
            pltpu.make_async_remote_copy(
                src_ref=comm_ref.at[my],
                dst_ref=comm_ref.at[my],
                send_sem=send_sems.at[peer],
                recv_sem=recv_sems.at[peer],
                device_id=(peer,),
                device_id_type=pl.DeviceIdType.MESH,
            ).wait_send()

    return pl.pallas_call(
        body,
        out_shape=jax.ShapeDtypeStruct((2, n), jnp.float32),
        in_specs=[pl.BlockSpec(memory_space=pl.ANY)],
        out_specs=pl.BlockSpec(memory_space=pltpu.VMEM),
        scratch_shapes=[
            pltpu.VMEM((C, rows_per, n), jnp.float32),
            pltpu.VMEM((N_DEV, 2, n), jnp.float32),
            pltpu.SemaphoreType.DMA((C,)),
            pltpu.SemaphoreType.DMA((N_DEV,)),
            pltpu.SemaphoreType.DMA((N_DEV,)),
        ],
        compiler_params=pltpu.CompilerParams(collective_id=0),
    )(x)


# device time: 8869 ns/iter; 1.6016x vs baseline; 1.2122x over previous
import os

import jax
import jax.numpy as jnp
from jax import lax
from jax.experimental import pallas as pl
from jax.experimental.pallas import tpu as pltpu

N_DEV = 8

_VARIANT = os.environ.get("KERNEL_VARIANT", "full")


def kernel(x):
    m_per, n = x.shape

    if _VARIANT == "copyonly":
        def body_copy(x_ref, out_ref):
            out_ref[:, :] = x_ref[:2, :]
        return pl.pallas_call(
            body_copy,
            out_shape=jax.ShapeDtypeStruct((2, n), jnp.float32),
            in_specs=[pl.BlockSpec(memory_space=pltpu.VMEM)],
            out_specs=pl.BlockSpec(memory_space=pltpu.VMEM),
        )(x)

    if _VARIANT == "barrieronly":
        def body_bo(x_ref, out_ref):
            my = lax.axis_index("i")
            barrier_sem = pltpu.get_barrier_semaphore()
            for d in range(1, N_DEV):
                peer = lax.rem(my + d, N_DEV)
                pl.semaphore_signal(
                    barrier_sem, inc=1,
                    device_id=(peer,), device_id_type=pl.DeviceIdType.MESH,
                )
            pl.semaphore_wait(barrier_sem, N_DEV - 1)
            out_ref[0, :] = jnp.ones((n,), jnp.float32)
            out_ref[1, :] = jnp.ones((n,), jnp.float32)
        return pl.pallas_call(
            body_bo,
            out_shape=jax.ShapeDtypeStruct((2, n), jnp.float32),
            in_specs=[pl.BlockSpec(memory_space=pl.ANY)],
            out_specs=pl.BlockSpec(memory_space=pltpu.VMEM),
            compiler_params=pltpu.CompilerParams(collective_id=0),
        )(x)

    if _VARIANT == "commonly":
        def body_co(x_ref, out_ref, comm_ref, send_sems, recv_sems):
            my = lax.axis_index("i")
            barrier_sem = pltpu.get_barrier_semaphore()
            for d in range(1, N_DEV):
                peer = lax.rem(my + d, N_DEV)
                pl.semaphore_signal(
                    barrier_sem, inc=1,
                    device_id=(peer,), device_id_type=pl.DeviceIdType.MESH,
                )
            comm_ref[my, 0, :] = jnp.ones((n,), jnp.float32)
            comm_ref[my, 1, :] = jnp.ones((n,), jnp.float32)
            pl.semaphore_wait(barrier_sem, N_DEV - 1)
            for d in range(1, N_DEV):
                peer = lax.rem(my + d, N_DEV)
                pltpu.make_async_remote_copy(
                    src_ref=comm_ref.at[my], dst_ref=comm_ref.at[my],
                    send_sem=send_sems.at[peer], recv_sem=recv_sems.at[my],
                    device_id=(peer,), device_id_type=pl.DeviceIdType.MESH,
                ).start()
            for d in range(1, N_DEV):
                peer = lax.rem(my + d, N_DEV)
                pltpu.make_async_remote_copy(
                    src_ref=comm_ref.at[peer], dst_ref=comm_ref.at[peer],
                    send_sem=send_sems.at[peer], recv_sem=recv_sems.at[peer],
                    device_id=(peer,), device_id_type=pl.DeviceIdType.MESH,
                ).wait_recv()
            vals = comm_ref[:, 0, :]
            idxs = comm_ref[:, 1, :]
            gmax = jnp.max(vals, axis=0)
            gidx = jnp.min(
                jnp.where(vals == gmax[None, :], idxs, jnp.float32(jnp.inf)),
                axis=0,
            )
            out_ref[0, :] = gmax
            out_ref[1, :] = gidx
            for d in range(1, N_DEV):
                peer = lax.rem(my + d, N_DEV)
                pltpu.make_async_remote_copy(
                    src_ref=comm_ref.at[my], dst_ref=comm_ref.at[my],
                    send_sem=send_sems.at[peer], recv_sem=recv_sems.at[peer],
                    device_id=(peer,), device_id_type=pl.DeviceIdType.MESH,
                ).wait_send()
        return pl.pallas_call(
            body_co,
            out_shape=jax.ShapeDtypeStruct((2, n), jnp.float32),
            in_specs=[pl.BlockSpec(memory_space=pl.ANY)],
            out_specs=pl.BlockSpec(memory_space=pltpu.VMEM),
            scratch_shapes=[
                pltpu.VMEM((N_DEV, 2, n), jnp.float32),
                pltpu.SemaphoreType.DMA((N_DEV,)),
                pltpu.SemaphoreType.DMA((N_DEV,)),
            ],
            compiler_params=pltpu.CompilerParams(collective_id=0),
        )(x)

    if _VARIANT == "nocomm":
        def body_nc(x_ref, out_ref):
            my = lax.axis_index("i")
            n_blk = m_per // 8

            def step(b, carry):
                m, bidx = carry
                blk = x_ref[pl.ds(b * 8, 8), :]
                take = blk > m
                return (jnp.where(take, blk, m), jnp.where(take, b, bidx))

            m0 = jnp.full((8, n), -jnp.inf, jnp.float32)
            b0 = jnp.zeros((8, n), jnp.int32)
            m, bidx = lax.fori_loop(0, n_blk, step, (m0, b0), unroll=8)
            local_max = jnp.max(m, axis=0)
            sub = lax.broadcasted_iota(jnp.int32, (8, n), 0)
            rows = bidx * 8 + sub
            cand = jnp.where(m == local_max[None, :], rows, jnp.int32(m_per))
            out_ref[0, :] = local_max
            out_ref[1, :] = jnp.min(cand, axis=0).astype(jnp.float32) + (
                my.astype(jnp.float32) * jnp.float32(m_per)
            )
        return pl.pallas_call(
            body_nc,
            out_shape=jax.ShapeDtypeStruct((2, n), jnp.float32),
            in_specs=[pl.BlockSpec(memory_space=pltpu.VMEM)],
            out_specs=pl.BlockSpec(memory_space=pltpu.VMEM),
        )(x)

    C = 4
    rows_per = m_per // C

    def body(x_ref, out_ref, xbuf, comm_ref, copy_sems, send_sems, recv_sems):
        my = lax.axis_index("i")

        copies = []
        for c in range(C):
            cp = pltpu.make_async_copy(
                x_ref.at[pl.ds(c * rows_per, rows_per), :],
                xbuf.at[c],
                copy_sems.at[c],
            )
            cp.start()
            copies.append(cp)

        barrier_sem = pltpu.get_barrier_semaphore()
        for d in range(1, N_DEV):
            peer = lax.rem(my + d, N_DEV)
            pl.semaphore_signal(
                barrier_sem, inc=1,
                device_id=(peer,), device_id_type=pl.DeviceIdType.MESH,
            )

        m = jnp.full((8, n), -jnp.inf, jnp.float32)
        bidx = jnp.zeros((8, n), jnp.int32)
        for c in range(C):
            copies[c].wait()

            def step(b, carry, c=c):
                mm, bb = carry
                blk = xbuf[c, pl.ds(b * 8, 8), :]
                take = blk > mm
                return (
                    jnp.where(take, blk, mm),
                    jnp.where(take, b + c * (rows_per // 8), bb),
                )

            m, bidx = lax.fori_loop(
                0, rows_per // 8, step, (m, bidx), unroll=8
            )

        local_max = jnp.max(m, axis=0)
        sub = lax.broadcasted_iota(jnp.int32, (8, n), 0)
        rows = bidx * 8 + sub
        cand = jnp.where(m == local_max[None, :], rows, jnp.int32(m_per))
        local_idx = jnp.min(cand, axis=0).astype(jnp.float32) + (
            my.astype(jnp.float32) * jnp.float32(m_per)
        )
        comm_ref[my, 0, :] = local_max
        comm_ref[my, 1, :] = local_idx

        pl.semaphore_wait(barrier_sem, N_DEV - 1)

        for d in range(1, N_DEV):
            peer = lax.rem(my + d, N_DEV)
            pltpu.make_async_remote_copy(
                src_ref=comm_ref.at[my],
                dst_ref=comm_ref.at[my],
                send_sem=send_sems.at[peer],
                recv_sem=recv_sems.at[my],
                device_id=(peer,),
                device_id_type=pl.DeviceIdType.MESH,
            ).start()

        for d in range(1, N_DEV):
            peer = lax.rem(my + d, N_DEV)
            pltpu.make_async_remote_copy(
                src_ref=comm_ref.at[peer],
                dst_ref=comm_ref.at[peer],
                send_sem=send_sems.at[peer],
                recv_sem=recv_sems.at[peer],
                device_id=(peer,),
                device_id_type=pl.DeviceIdType.MESH,
            ).wait_recv()

        vals = comm_ref[:, 0, :]
        idxs = comm_ref[:, 1, :]
        gmax = jnp.max(vals, axis=0)
        gidx = jnp.min(
            jnp.where(vals == gmax[None, :], idxs, jnp.float32(jnp.inf)), axis=0
        )
        out_ref[0, :] = gmax
        out_ref[1, :] = gidx

        for d in range(1, N_DEV):
            peer = lax.rem(my + d, N_DEV)
            pltpu.make_async_remote_copy(
                src_ref=comm_ref.at[my],
                dst_ref=comm_ref.at[my],
                send_sem=send_sems.at[peer],
                recv_sem=recv_sems.at[peer],
                device_id=(peer,),
                device_id_type=pl.DeviceIdType.MESH,
            ).wait_send()

    return pl.pallas_call(
        body,
        out_shape=jax.ShapeDtypeStruct((2, n), jnp.float32),
        in_specs=[pl.BlockSpec(memory_space=pl.ANY)],
        out_specs=pl.BlockSpec(memory_space=pltpu.VMEM),
        scratch_shapes=[
            pltpu.VMEM((C, rows_per, n), jnp.float32),
            pltpu.VMEM((N_DEV, 2, n), jnp.float32),
            pltpu.SemaphoreType.DMA((C,)),
            pltpu.SemaphoreType.DMA((N_DEV,)),
            pltpu.SemaphoreType.DMA((N_DEV,)),
        ],
        compiler_params=pltpu.CompilerParams(collective_id=0),
    )(x)
